# baseline (speedup 1.0000x reference)
import functools

import jax
import jax.numpy as jnp
from jax.experimental import pallas as pl
from jax.experimental.pallas import tpu as pltpu

N_EXP = 64
K = 2


def _router_kernel(x_ref, w_ref, idx_ref, wts_ref, cnt_ref, psum_ref, aux_ref,
                   *, n_tokens, n_steps):
    step = pl.program_id(0)

    logits = jax.lax.dot_general(
        w_ref[...], x_ref[...],
        (((1,), (1,)), ((), ())),
        preferred_element_type=jnp.float32)

    iota = jax.lax.broadcasted_iota(jnp.int32, logits.shape, 0)

    rowmax = jnp.max(logits, axis=0, keepdims=True)
    i1 = jnp.min(jnp.where(logits == rowmax, iota, N_EXP),
                 axis=0, keepdims=True)
    hit1 = iota == i1
    masked = jnp.where(hit1, -jnp.inf, logits)
    m2 = jnp.max(masked, axis=0, keepdims=True)
    i2 = jnp.min(jnp.where(masked == m2, iota, N_EXP),
                 axis=0, keepdims=True)
    hit2 = iota == i2

    ex = jnp.exp(logits - rowmax)
    denom = jnp.sum(ex, axis=0, keepdims=True)
    rdenom = 1.0 / denom
    probs = ex * rdenom
    p1 = rdenom
    p2 = jnp.exp(m2 - rowmax) * rdenom

    s = p1 + p2 + 1e-8
    parity = jax.lax.broadcasted_iota(jnp.int32, idx_ref.shape, 1) & 1
    idx_ref[...] = jnp.where(parity == 0,
                             jnp.repeat(i1, 2, axis=1),
                             jnp.repeat(i2, 2, axis=1))
    wts_ref[...] = jnp.where(parity == 0,
                             jnp.repeat(p1 / s, 2, axis=1),
                             jnp.repeat(p2 / s, 2, axis=1))

    cnt_blk = jnp.sum(hit1.astype(jnp.float32) + hit2.astype(jnp.float32),
                      axis=1, keepdims=True)
    psum_blk = jnp.sum(probs, axis=1, keepdims=True)

    @pl.when(step == 0)
    def _init():
        cnt_ref[...] = cnt_blk
        psum_ref[...] = psum_blk

    @pl.when(step != 0)
    def _acc():
        cnt_ref[...] += cnt_blk
        psum_ref[...] += psum_blk

    @pl.when(step == n_steps - 1)
    def _finalize():
        f = cnt_ref[...] / (n_tokens * K)
        p = psum_ref[...] / n_tokens
        aux_ref[...] = (N_EXP * jnp.sum(f * p)).reshape(1, 1)


def kernel(x, W):
    b, s, d = x.shape
    n_tokens = b * s
    x_flat = x.reshape(n_tokens, d)

    block_t = 2048
    n_steps = n_tokens // block_t

    grid_spec = pl.GridSpec(
        grid=(n_steps,),
        in_specs=[
            pl.BlockSpec((block_t, d), lambda i: (i, 0)),
            pl.BlockSpec((N_EXP, d), lambda i: (0, 0)),
        ],
        out_specs=[
            pl.BlockSpec((1, K * block_t), lambda i: (0, i)),
            pl.BlockSpec((1, K * block_t), lambda i: (0, i)),
            pl.BlockSpec((N_EXP, 1), lambda i: (0, 0)),
            pl.BlockSpec((N_EXP, 1), lambda i: (0, 0)),
            pl.BlockSpec((1, 1), lambda i: (0, 0)),
        ],
    )

    idx_t, wts_t, _cnt, _psum, aux = pl.pallas_call(
        functools.partial(_router_kernel, n_tokens=n_tokens, n_steps=n_steps),
        grid_spec=grid_spec,
        out_shape=[
            jax.ShapeDtypeStruct((1, K * n_tokens), jnp.int32),
            jax.ShapeDtypeStruct((1, K * n_tokens), jnp.float32),
            jax.ShapeDtypeStruct((N_EXP, 1), jnp.float32),
            jax.ShapeDtypeStruct((N_EXP, 1), jnp.float32),
            jax.ShapeDtypeStruct((1, 1), jnp.float32),
        ],
        compiler_params=pltpu.CompilerParams(
            dimension_semantics=("arbitrary",),
        ),
    )(x_flat, W)

    return (idx_t.reshape(n_tokens, K), wts_t.reshape(n_tokens, K),
            aux[0, 0])


# speedup vs baseline: 3.0796x; 3.0796x over previous
import functools

import jax
import jax.numpy as jnp
from jax.experimental import pallas as pl
from jax.experimental.pallas import tpu as pltpu

N_EXP = 64
K = 2


def _router_kernel(x_ref, w_ref, idx_ref, wts_ref, cnt_ref, psum_ref, aux_ref,
                   *, n_tokens, n_steps):
    step = pl.program_id(0)

    logits = jax.lax.dot_general(
        w_ref[...], x_ref[...],
        (((1,), (1,)), ((), ())),
        preferred_element_type=jnp.float32)

    iota = jax.lax.broadcasted_iota(jnp.int32, logits.shape, 0)

    rowmax = jnp.max(logits, axis=0, keepdims=True)
    i1 = jnp.min(jnp.where(logits == rowmax, iota, N_EXP),
                 axis=0, keepdims=True)
    hit1 = iota == i1
    masked = jnp.where(hit1, -jnp.inf, logits)
    m2 = jnp.max(masked, axis=0, keepdims=True)
    i2 = jnp.min(jnp.where(masked == m2, iota, N_EXP),
                 axis=0, keepdims=True)
    hit2 = iota == i2

    ex = jnp.exp(logits - rowmax)
    denom = jnp.sum(ex, axis=0, keepdims=True)
    rdenom = 1.0 / denom
    probs = ex * rdenom
    p1 = rdenom
    p2 = jnp.exp(m2 - rowmax) * rdenom

    s = p1 + p2 + 1e-8
    idx_ref[...] = jnp.concatenate([i1, i2], axis=0)
    wts_ref[...] = jnp.concatenate([p1 / s, p2 / s], axis=0)

    cnt_blk = jnp.sum(hit1.astype(jnp.float32) + hit2.astype(jnp.float32),
                      axis=1, keepdims=True)
    psum_blk = jnp.sum(probs, axis=1, keepdims=True)

    @pl.when(step == 0)
    def _init():
        cnt_ref[...] = cnt_blk
        psum_ref[...] = psum_blk

    @pl.when(step != 0)
    def _acc():
        cnt_ref[...] += cnt_blk
        psum_ref[...] += psum_blk

    @pl.when(step == n_steps - 1)
    def _finalize():
        f = cnt_ref[...] / (n_tokens * K)
        p = psum_ref[...] / n_tokens
        aux_ref[...] = (N_EXP * jnp.sum(f * p)).reshape(1, 1)


def kernel(x, W):
    b, s, d = x.shape
    n_tokens = b * s
    x_flat = x.reshape(n_tokens, d)

    block_t = 2048
    n_steps = n_tokens // block_t

    grid_spec = pl.GridSpec(
        grid=(n_steps,),
        in_specs=[
            pl.BlockSpec((block_t, d), lambda i: (i, 0)),
            pl.BlockSpec((N_EXP, d), lambda i: (0, 0)),
        ],
        out_specs=[
            pl.BlockSpec((K, block_t), lambda i: (0, i)),
            pl.BlockSpec((K, block_t), lambda i: (0, i)),
            pl.BlockSpec((N_EXP, 1), lambda i: (0, 0)),
            pl.BlockSpec((N_EXP, 1), lambda i: (0, 0)),
            pl.BlockSpec((1, 1), lambda i: (0, 0)),
        ],
    )

    idx_t, wts_t, _cnt, _psum, aux = pl.pallas_call(
        functools.partial(_router_kernel, n_tokens=n_tokens, n_steps=n_steps),
        grid_spec=grid_spec,
        out_shape=[
            jax.ShapeDtypeStruct((K, n_tokens), jnp.int32),
            jax.ShapeDtypeStruct((K, n_tokens), jnp.float32),
            jax.ShapeDtypeStruct((N_EXP, 1), jnp.float32),
            jax.ShapeDtypeStruct((N_EXP, 1), jnp.float32),
            jax.ShapeDtypeStruct((1, 1), jnp.float32),
        ],
        compiler_params=pltpu.CompilerParams(
            dimension_semantics=("arbitrary",),
        ),
    )(x_flat, W)

    return (idx_t.T, wts_t.T, aux[0, 0])
